# Initial kernel scaffold; baseline (speedup 1.0000x reference)
import jax
import jax.numpy as jnp
from jax import lax
from jax.experimental import pallas as pl
from jax.experimental.pallas import tpu as pltpu

N = 20000
R = 160
PAD = R * 128
K_PRE = 6000
K_POST = 1000
THR = 0.7


def _nms_body(s_ref, x1_ref, y1_ref, x2_ref, y2_ref,
              ox1, oy1, ox2, oy2, osc,
              ms_ref, area_ref, lin_ref):
    s = s_ref[...]
    u = lax.bitcast_convert_type(s, jnp.uint32)
    key = jnp.where(s < 0, ~u, u | jnp.uint32(0x80000000))

    prefix = jnp.uint32(0)
    for b in range(31, -1, -1):
        cand = prefix | jnp.uint32(1 << b)
        cnt = jnp.sum((key >= cand).astype(jnp.int32))
        prefix = jnp.where(cnt >= K_PRE, cand, prefix)

    cnt_gt = jnp.sum((key > prefix).astype(jnp.int32))
    tie = key == prefix
    tie_i = tie.astype(jnp.int32)
    tcum = jnp.cumsum(tie_i, axis=1)
    row_tot = tcum[:, 127:128]
    row_off = jnp.cumsum(row_tot, axis=0) - row_tot
    ordinal = row_off + tcum - tie_i
    is_cand = (key > prefix) | (tie & (ordinal < (K_PRE - cnt_gt)))

    ms_ref[...] = jnp.where(is_cand, s, -jnp.inf)
    x1 = x1_ref[...]
    y1 = y1_ref[...]
    x2 = x2_ref[...]
    y2 = y2_ref[...]
    area_ref[...] = jnp.maximum(x2 - x1, 0.0) * jnp.maximum(y2 - y1, 0.0)
    lin_ref[...] = (lax.broadcasted_iota(jnp.int32, (R, 128), 0) * 128
                    + lax.broadcasted_iota(jnp.int32, (R, 128), 1))

    def step(i, carry):
        ms = ms_ref[...]
        m = jnp.max(ms)
        valid = m > -jnp.inf
        lin = lin_ref[...]
        eq = ms == m
        j = jnp.min(jnp.where(eq, lin, jnp.int32(2 ** 30)))
        isj = lin == j
        x1 = x1_ref[...]
        y1 = y1_ref[...]
        x2 = x2_ref[...]
        y2 = y2_ref[...]
        area = area_ref[...]
        bx1 = jnp.sum(jnp.where(isj, x1, 0.0))
        by1 = jnp.sum(jnp.where(isj, y1, 0.0))
        bx2 = jnp.sum(jnp.where(isj, x2, 0.0))
        by2 = jnp.sum(jnp.where(isj, y2, 0.0))
        ba = jnp.sum(jnp.where(isj, area, 0.0))
        xx1 = jnp.maximum(bx1, x1)
        yy1 = jnp.maximum(by1, y1)
        xx2 = jnp.minimum(bx2, x2)
        yy2 = jnp.minimum(by2, y2)
        inter = jnp.maximum(xx2 - xx1, 0.0) * jnp.maximum(yy2 - yy1, 0.0)
        iou = inter / (ba + area - inter + 1e-9)
        sup = (iou > THR) | isj
        ms_ref[...] = jnp.where(jnp.logical_and(valid, sup), -jnp.inf, ms)
        ox1[i, 0] = jnp.where(valid, bx1, 0.0)
        oy1[i, 0] = jnp.where(valid, by1, 0.0)
        ox2[i, 0] = jnp.where(valid, bx2, 0.0)
        oy2[i, 0] = jnp.where(valid, by2, 0.0)
        osc[i, 0] = jnp.where(valid, m, 0.0)
        return carry

    lax.fori_loop(0, K_POST, step, 0)


def kernel(boxes, scores):
    s = jnp.pad(scores, (0, PAD - N), constant_values=-jnp.inf).reshape(R, 128)
    bx = jnp.pad(boxes, ((0, PAD - N), (0, 0)))
    x1 = bx[:, 0].reshape(R, 128)
    y1 = bx[:, 1].reshape(R, 128)
    x2 = bx[:, 2].reshape(R, 128)
    y2 = bx[:, 3].reshape(R, 128)

    smem = pl.BlockSpec(memory_space=pltpu.MemorySpace.SMEM)
    outs = pl.pallas_call(
        _nms_body,
        out_shape=[jax.ShapeDtypeStruct((K_POST, 1), jnp.float32)] * 5,
        out_specs=[smem] * 5,
        scratch_shapes=[
            pltpu.VMEM((R, 128), jnp.float32),
            pltpu.VMEM((R, 128), jnp.float32),
            pltpu.VMEM((R, 128), jnp.int32),
        ],
    )(s, x1, y1, x2, y2)
    return jnp.concatenate(outs, axis=1)


# speedup vs baseline: 15.9536x; 15.9536x over previous
import jax
import jax.numpy as jnp
from jax import lax
from jax.experimental import pallas as pl
from jax.experimental.pallas import tpu as pltpu

N = 20000
R = 160
PAD = R * 128
K_PRE = 6000
K_POST = 1000
THR = 0.7


def _nms_body(s_ref, x1_ref, y1_ref, x2_ref, y2_ref,
              ox1, oy1, ox2, oy2, osc,
              ms_ref, area_ref, lin_ref):
    s = s_ref[...]
    u = lax.bitcast_convert_type(s, jnp.uint32)
    key = jnp.where(s < 0, ~u, u | jnp.uint32(0x80000000))

    prefix = jnp.uint32(0)
    for b in range(31, -1, -1):
        cand = prefix | jnp.uint32(1 << b)
        cnt = jnp.sum((key >= cand).astype(jnp.int32))
        prefix = jnp.where(cnt >= K_PRE, cand, prefix)

    cnt_gt = jnp.sum((key > prefix).astype(jnp.int32))
    tie = key == prefix
    tie_f = tie.astype(jnp.float32)
    incl = (lax.broadcasted_iota(jnp.int32, (128, 128), 0)
            <= lax.broadcasted_iota(jnp.int32, (128, 128), 1)).astype(jnp.float32)
    tcum = jnp.dot(tie_f, incl, preferred_element_type=jnp.float32)
    row_tot = tcum[:, 127:128]
    strict = (lax.broadcasted_iota(jnp.int32, (R, R), 1)
              < lax.broadcasted_iota(jnp.int32, (R, R), 0)).astype(jnp.float32)
    row_off = jnp.dot(strict, row_tot, preferred_element_type=jnp.float32)
    ordinal = row_off + tcum - tie_f
    need = (K_PRE - cnt_gt).astype(jnp.float32)
    is_cand = (key > prefix) | (tie & (ordinal < need))

    ms_ref[...] = jnp.where(is_cand, s, -jnp.inf)
    x1 = x1_ref[...]
    y1 = y1_ref[...]
    x2 = x2_ref[...]
    y2 = y2_ref[...]
    area_ref[...] = jnp.maximum(x2 - x1, 0.0) * jnp.maximum(y2 - y1, 0.0)
    lin_ref[...] = (lax.broadcasted_iota(jnp.int32, (R, 128), 0) * 128
                    + lax.broadcasted_iota(jnp.int32, (R, 128), 1))

    def step(i, carry):
        ms = ms_ref[...]
        m = jnp.max(ms)
        valid = m > -jnp.inf
        lin = lin_ref[...]
        eq = ms == m
        j = jnp.min(jnp.where(eq, lin, jnp.int32(2 ** 30)))
        isj = lin == j
        x1 = x1_ref[...]
        y1 = y1_ref[...]
        x2 = x2_ref[...]
        y2 = y2_ref[...]
        area = area_ref[...]
        bx1 = jnp.sum(jnp.where(isj, x1, 0.0))
        by1 = jnp.sum(jnp.where(isj, y1, 0.0))
        bx2 = jnp.sum(jnp.where(isj, x2, 0.0))
        by2 = jnp.sum(jnp.where(isj, y2, 0.0))
        ba = jnp.sum(jnp.where(isj, area, 0.0))
        xx1 = jnp.maximum(bx1, x1)
        yy1 = jnp.maximum(by1, y1)
        xx2 = jnp.minimum(bx2, x2)
        yy2 = jnp.minimum(by2, y2)
        inter = jnp.maximum(xx2 - xx1, 0.0) * jnp.maximum(yy2 - yy1, 0.0)
        iou = inter / (ba + area - inter + 1e-9)
        sup = (iou > THR) | isj
        ms_ref[...] = jnp.where(jnp.logical_and(valid, sup), -jnp.inf, ms)
        ox1[pl.ds(i, 1), :] = jnp.reshape(jnp.where(valid, bx1, 0.0), (1, 1))
        oy1[pl.ds(i, 1), :] = jnp.reshape(jnp.where(valid, by1, 0.0), (1, 1))
        ox2[pl.ds(i, 1), :] = jnp.reshape(jnp.where(valid, bx2, 0.0), (1, 1))
        oy2[pl.ds(i, 1), :] = jnp.reshape(jnp.where(valid, by2, 0.0), (1, 1))
        osc[pl.ds(i, 1), :] = jnp.reshape(jnp.where(valid, m, 0.0), (1, 1))
        return carry

    lax.fori_loop(0, K_POST, step, 0)


def kernel(boxes, scores):
    s = jnp.pad(scores, (0, PAD - N), constant_values=-jnp.inf).reshape(R, 128)
    bx = jnp.pad(boxes, ((0, PAD - N), (0, 0)))
    x1 = bx[:, 0].reshape(R, 128)
    y1 = bx[:, 1].reshape(R, 128)
    x2 = bx[:, 2].reshape(R, 128)
    y2 = bx[:, 3].reshape(R, 128)

    outs = pl.pallas_call(
        _nms_body,
        out_shape=[jax.ShapeDtypeStruct((K_POST, 1), jnp.float32)] * 5,
        scratch_shapes=[
            pltpu.VMEM((R, 128), jnp.float32),
            pltpu.VMEM((R, 128), jnp.float32),
            pltpu.VMEM((R, 128), jnp.int32),
        ],
    )(s, x1, y1, x2, y2)
    return jnp.concatenate(outs, axis=1)


# speedup vs baseline: 16.7292x; 1.0486x over previous
import jax
import jax.numpy as jnp
from jax import lax
from jax.experimental import pallas as pl
from jax.experimental.pallas import tpu as pltpu

N = 20000
R = 160
PAD = R * 128
K_PRE = 6000
K_POST = 1000
THR = 0.7


def _nms_body(s_ref, x1_ref, y1_ref, x2_ref, y2_ref,
              ox1, oy1, ox2, oy2, osc,
              ms_ref, area_ref, lin_ref):
    s = s_ref[...]
    u = lax.bitcast_convert_type(s, jnp.uint32)
    key = jnp.where(s < 0, ~u, u | jnp.uint32(0x80000000))

    prefix = jnp.uint32(0)
    for b in range(31, -1, -1):
        cand = prefix | jnp.uint32(1 << b)
        cnt = jnp.sum((key >= cand).astype(jnp.int32))
        prefix = jnp.where(cnt >= K_PRE, cand, prefix)

    cnt_gt = jnp.sum((key > prefix).astype(jnp.int32))
    tie = key == prefix
    tie_f = tie.astype(jnp.float32)
    incl = (lax.broadcasted_iota(jnp.int32, (128, 128), 0)
            <= lax.broadcasted_iota(jnp.int32, (128, 128), 1)).astype(jnp.float32)
    tcum = jnp.dot(tie_f, incl, preferred_element_type=jnp.float32)
    row_tot = tcum[:, 127:128]
    strict = (lax.broadcasted_iota(jnp.int32, (R, R), 1)
              < lax.broadcasted_iota(jnp.int32, (R, R), 0)).astype(jnp.float32)
    row_off = jnp.dot(strict, row_tot, preferred_element_type=jnp.float32)
    ordinal = row_off + tcum - tie_f
    need = (K_PRE - cnt_gt).astype(jnp.float32)
    is_cand = (key > prefix) | (tie & (ordinal < need))

    ms_ref[...] = jnp.where(is_cand, s, -jnp.inf)
    x1 = x1_ref[...]
    y1 = y1_ref[...]
    x2 = x2_ref[...]
    y2 = y2_ref[...]
    area_ref[...] = jnp.maximum(x2 - x1, 0.0) * jnp.maximum(y2 - y1, 0.0)
    lin_ref[...] = (lax.broadcasted_iota(jnp.int32, (R, 128), 0) * 128
                    + lax.broadcasted_iota(jnp.int32, (R, 128), 1))

    def step(i, carry):
        ms = ms_ref[...]
        m = jnp.max(ms)
        valid = m > -jnp.inf
        lin = lin_ref[...]
        eq = ms == m
        j = jnp.min(jnp.where(eq, lin, jnp.int32(2 ** 30)))
        isj = lin == j
        r = j >> 7
        c = j & 127
        x1 = x1_ref[...]
        y1 = y1_ref[...]
        x2 = x2_ref[...]
        y2 = y2_ref[...]
        area = area_ref[...]
        lane = lax.broadcasted_iota(jnp.int32, (1, 128), 1) == c

        def pick(ref):
            return jnp.sum(jnp.where(lane, ref[pl.ds(r, 1), :], 0.0))

        bx1 = pick(x1_ref)
        by1 = pick(y1_ref)
        bx2 = pick(x2_ref)
        by2 = pick(y2_ref)
        ba = pick(area_ref)
        xx1 = jnp.maximum(bx1, x1)
        yy1 = jnp.maximum(by1, y1)
        xx2 = jnp.minimum(bx2, x2)
        yy2 = jnp.minimum(by2, y2)
        inter = jnp.maximum(xx2 - xx1, 0.0) * jnp.maximum(yy2 - yy1, 0.0)
        iou = inter / (ba + area - inter + 1e-9)
        sup = (iou > THR) | isj
        ms_ref[...] = jnp.where(jnp.logical_and(valid, sup), -jnp.inf, ms)
        ox1[pl.ds(i, 1), :] = jnp.reshape(jnp.where(valid, bx1, 0.0), (1, 1))
        oy1[pl.ds(i, 1), :] = jnp.reshape(jnp.where(valid, by1, 0.0), (1, 1))
        ox2[pl.ds(i, 1), :] = jnp.reshape(jnp.where(valid, bx2, 0.0), (1, 1))
        oy2[pl.ds(i, 1), :] = jnp.reshape(jnp.where(valid, by2, 0.0), (1, 1))
        osc[pl.ds(i, 1), :] = jnp.reshape(jnp.where(valid, m, 0.0), (1, 1))
        return carry

    lax.fori_loop(0, K_POST, step, 0)


def kernel(boxes, scores):
    s = jnp.pad(scores, (0, PAD - N), constant_values=-jnp.inf).reshape(R, 128)
    bx = jnp.pad(boxes, ((0, PAD - N), (0, 0)))
    x1 = bx[:, 0].reshape(R, 128)
    y1 = bx[:, 1].reshape(R, 128)
    x2 = bx[:, 2].reshape(R, 128)
    y2 = bx[:, 3].reshape(R, 128)

    outs = pl.pallas_call(
        _nms_body,
        out_shape=[jax.ShapeDtypeStruct((K_POST, 1), jnp.float32)] * 5,
        scratch_shapes=[
            pltpu.VMEM((R, 128), jnp.float32),
            pltpu.VMEM((R, 128), jnp.float32),
            pltpu.VMEM((R, 128), jnp.int32),
        ],
    )(s, x1, y1, x2, y2)
    return jnp.concatenate(outs, axis=1)
